# Initial kernel scaffold; baseline (speedup 1.0000x reference)
import jax
import jax.numpy as jnp
from jax.experimental import pallas as pl


def kernel(x, ln1_scale, ln1_bias, Wqkv, Wout, ln2_scale, ln2_bias, Wg, W1, W2, residual_scale):
    raise NotImplementedError("write your pallas kernel here")


# speedup vs baseline: 2.7087x; 2.7087x over previous
import functools

import jax
import jax.numpy as jnp
from jax.experimental import pallas as pl
from jax.experimental.pallas import tpu as pltpu

S = 2048
D = 768
H = 12
DK = 64
DFF = 3072
E = 8
EPAD = 128
TOPK = 2
LN_EPS = 1e-5

_F32 = jnp.float32
_BF16 = jnp.bfloat16


def _dot_t(a, b):
    return jax.lax.dot_general(a, b, (((1,), (1,)), ((), ())),
                               preferred_element_type=_F32)


def _ln_qkv_kernel(x_ref, g_ref, b_ref, w_ref, q_ref, k_ref, v_ref):
    x = x_ref[...]
    m = jnp.mean(x, axis=-1, keepdims=True)
    v = jnp.mean((x - m) * (x - m), axis=-1, keepdims=True)
    nx = (x - m) / jnp.sqrt(v + LN_EPS) * g_ref[0:1, :] + b_ref[0:1, :]
    qkv = _dot_t(nx.astype(_BF16), w_ref[...])
    q_ref[...] = qkv[:, 0:D].astype(_BF16)
    k_ref[...] = qkv[:, D:2 * D].astype(_BF16)
    v_ref[...] = qkv[:, 2 * D:3 * D].astype(_BF16)


def _ln_qkv(x, g, b, w_bf):
    bt = 256
    return pl.pallas_call(
        _ln_qkv_kernel,
        grid=(S // bt,),
        in_specs=[
            pl.BlockSpec((bt, D), lambda i: (i, 0)),
            pl.BlockSpec((8, D), lambda i: (0, 0)),
            pl.BlockSpec((8, D), lambda i: (0, 0)),
            pl.BlockSpec((3 * D, D), lambda i: (0, 0)),
        ],
        out_specs=[pl.BlockSpec((bt, D), lambda i: (i, 0))] * 3,
        out_shape=[jax.ShapeDtypeStruct((S, D), _BF16)] * 3,
        compiler_params=pltpu.CompilerParams(
            dimension_semantics=("arbitrary",)),
    )(x, g, b, w_bf)


def _attn_kernel(q_ref, k_ref, v_ref, o_ref):
    qv = q_ref[0]
    kv = k_ref[0]
    s = _dot_t(qv, kv) * (1.0 / float(DK) ** 0.5)
    m = jnp.max(s, axis=-1, keepdims=True)
    p = jnp.exp(s - m)
    l = jnp.sum(p, axis=-1, keepdims=True)
    ctx = jax.lax.dot_general(p.astype(_BF16), v_ref[0],
                              (((1,), (0,)), ((), ())),
                              preferred_element_type=_F32)
    o_ref[0] = (ctx / l).astype(_BF16)


def _attention(q3, k3, v3):
    bq = 512
    return pl.pallas_call(
        _attn_kernel,
        grid=(H, S // bq),
        in_specs=[
            pl.BlockSpec((1, bq, DK), lambda h, i: (h, i, 0)),
            pl.BlockSpec((1, S, DK), lambda h, i: (h, 0, 0)),
            pl.BlockSpec((1, S, DK), lambda h, i: (h, 0, 0)),
        ],
        out_specs=pl.BlockSpec((1, bq, DK), lambda h, i: (h, i, 0)),
        out_shape=jax.ShapeDtypeStruct((H, S, DK), _BF16),
        compiler_params=pltpu.CompilerParams(
            dimension_semantics=("arbitrary", "arbitrary")),
    )(q3, k3, v3)


def _proj_router_kernel(ctx_ref, x_ref, wout_ref, g_ref, b_ref, wg_ref,
                        x1_ref, flat_ref, logits_ref, ewm_ref, ent_ref):
    i = pl.program_id(0)
    attn = _dot_t(ctx_ref[...], wout_ref[...])
    x1 = x_ref[...] + attn
    x1_ref[...] = x1
    m = jnp.mean(x1, axis=-1, keepdims=True)
    va = jnp.mean((x1 - m) * (x1 - m), axis=-1, keepdims=True)
    nx2 = (x1 - m) / jnp.sqrt(va + LN_EPS) * g_ref[0:1, :] + b_ref[0:1, :]
    flat_ref[...] = nx2.astype(_BF16)
    logits = jax.lax.dot_general(nx2, wg_ref[...], (((1,), (1,)), ((), ())),
                                 preferred_element_type=_F32)
    logits_ref[...] = logits
    lane = jax.lax.broadcasted_iota(jnp.int32, logits.shape, 1)
    emask = lane < E
    lgm = jnp.where(emask, logits, -1e30)
    mx = jnp.max(lgm, axis=-1, keepdims=True)
    ex = jnp.where(emask, jnp.exp(lgm - mx), 0.0)
    p = ex / jnp.sum(ex, axis=-1, keepdims=True)
    sel = jnp.zeros_like(p)
    for e in range(E):
        pe = p[:, e:e + 1]
        gt = jnp.sum(jnp.where(emask & (p > pe), 1.0, 0.0),
                     axis=-1, keepdims=True)
        eq_lt = jnp.sum(jnp.where(emask & (p == pe) & (lane < e), 1.0, 0.0),
                        axis=-1, keepdims=True)
        is_sel = (gt + eq_lt) < TOPK
        sel = sel + jnp.where((lane == e) & is_sel, 1.0, 0.0)
    top2sum = jnp.sum(sel * p, axis=-1, keepdims=True)
    ewm_ref[...] = sel * p / top2sum
    logp = jnp.log(jnp.clip(p, 1e-6, None))
    ent_part = -jnp.sum(p * logp) / float(S)

    @pl.when(i == 0)
    def _():
        ent_ref[...] = jnp.zeros_like(ent_ref)

    ent_ref[...] += ent_part


def _proj_router(ctx, x, wout_bf, g2, b2, wg_pad):
    bt = 256
    return pl.pallas_call(
        _proj_router_kernel,
        grid=(S // bt,),
        in_specs=[
            pl.BlockSpec((bt, D), lambda i: (i, 0)),
            pl.BlockSpec((bt, D), lambda i: (i, 0)),
            pl.BlockSpec((D, D), lambda i: (0, 0)),
            pl.BlockSpec((8, D), lambda i: (0, 0)),
            pl.BlockSpec((8, D), lambda i: (0, 0)),
            pl.BlockSpec((EPAD, D), lambda i: (0, 0)),
        ],
        out_specs=[
            pl.BlockSpec((bt, D), lambda i: (i, 0)),
            pl.BlockSpec((bt, D), lambda i: (i, 0)),
            pl.BlockSpec((bt, EPAD), lambda i: (i, 0)),
            pl.BlockSpec((bt, EPAD), lambda i: (i, 0)),
            pl.BlockSpec((8, 128), lambda i: (0, 0)),
        ],
        out_shape=[
            jax.ShapeDtypeStruct((S, D), _F32),
            jax.ShapeDtypeStruct((S, D), _BF16),
            jax.ShapeDtypeStruct((S, EPAD), _F32),
            jax.ShapeDtypeStruct((S, EPAD), _F32),
            jax.ShapeDtypeStruct((8, 128), _F32),
        ],
        compiler_params=pltpu.CompilerParams(
            dimension_semantics=("arbitrary",)),
    )(ctx, x, wout_bf, g2, b2, wg_pad)


def _moe_kernel(flat_ref, w1_ref, w2_ref, ewm_ref, x1_ref, out_ref):
    e = pl.program_id(1)
    h = _dot_t(flat_ref[...], w1_ref[0])
    h = 0.5 * h * (1.0 + jax.lax.erf(h * (0.5 ** 0.5)))
    lane = jax.lax.broadcasted_iota(jnp.int32, ewm_ref.shape, 1)
    w = jnp.sum(jnp.where(lane == e, ewm_ref[...], 0.0), axis=-1,
                keepdims=True)
    h = (h * w).astype(_BF16)
    y = _dot_t(h, w2_ref[0])

    @pl.when(e == 0)
    def _():
        out_ref[...] = x1_ref[...]

    out_ref[...] += y


def _moe_dense(flat_bf, w1_bf, w2_bf, ewm, x1):
    bt = 512
    return pl.pallas_call(
        _moe_kernel,
        grid=(S // bt, E),
        in_specs=[
            pl.BlockSpec((bt, D), lambda t, e: (t, 0)),
            pl.BlockSpec((1, DFF, D), lambda t, e: (e, 0, 0)),
            pl.BlockSpec((1, D, DFF), lambda t, e: (e, 0, 0)),
            pl.BlockSpec((bt, EPAD), lambda t, e: (t, 0)),
            pl.BlockSpec((bt, D), lambda t, e: (t, 0)),
        ],
        out_specs=pl.BlockSpec((bt, D), lambda t, e: (t, 0)),
        out_shape=jax.ShapeDtypeStruct((S, D), _F32),
        compiler_params=pltpu.CompilerParams(
            dimension_semantics=("arbitrary", "arbitrary")),
    )(flat_bf, w1_bf, w2_bf, ewm, x1)


def kernel(x, ln1_scale, ln1_bias, Wqkv, Wout, ln2_scale, ln2_bias, Wg, W1,
           W2, residual_scale):
    x2d = x.reshape(S, D)
    rs = residual_scale[0]
    g1 = jnp.broadcast_to(ln1_scale[None, :], (8, D))
    b1 = jnp.broadcast_to(ln1_bias[None, :], (8, D))
    g2 = jnp.broadcast_to(ln2_scale[None, :], (8, D))
    b2 = jnp.broadcast_to(ln2_bias[None, :], (8, D))
    wqkv_bf = Wqkv.astype(_BF16)
    wout_bf = (Wout * rs).astype(_BF16)
    wg_pad = jnp.zeros((EPAD, D), _F32).at[:E].set(Wg)
    w1_bf = W1.astype(_BF16)
    w2_bf = (W2 * rs).astype(_BF16)

    q, k, v = _ln_qkv(x2d, g1, b1, wqkv_bf)

    def _heads(t):
        return t.reshape(S, H, DK).transpose(1, 0, 2)

    ctx3 = _attention(_heads(q), _heads(k), _heads(v))
    ctx = ctx3.transpose(1, 0, 2).reshape(S, D)
    x1, flat_bf, logits_pad, ewm, ent = _proj_router(
        ctx, x2d, wout_bf, g2, b2, wg_pad)
    out2d = _moe_dense(flat_bf, w1_bf, w2_bf, ewm, x1)

    out = out2d.reshape(1, S, D)
    router_logits = logits_pad[:, :E]
    entropy_loss = ent[0, 0]
    return (out, router_logits, entropy_loss)
